# baseline (speedup 1.0000x reference)
import functools

import jax
import jax.numpy as jnp
from jax import lax
from jax.experimental import pallas as pl
from jax.experimental.pallas import tpu as pltpu
from jax.experimental.pallas import tpu_sc as plsc

R, C = 64, 32768
B = 4096
L = 16

_info = plsc.get_sparse_core_info()
_NC, _NS = _info.num_cores, _info.num_subcores
NW = _NC * _NS
ROWS_PER_W = R // NW

_mesh = plsc.VectorSubcoreMesh(core_axis_name="c", subcore_axis_name="s")


@functools.partial(
    pl.kernel,
    mesh=_mesh,
    out_type=jax.ShapeDtypeStruct((R, B), jnp.float32),
    scratch_types=[
        [pltpu.VMEM((C,), jnp.float32) for _ in range(ROWS_PER_W)],
        pltpu.VMEM((ROWS_PER_W, B), jnp.int32),
        pltpu.VMEM((ROWS_PER_W, B), jnp.float32),
        [pltpu.SemaphoreType.DMA for _ in range(ROWS_PER_W)],
        pltpu.SemaphoreType.DMA,
        pltpu.SemaphoreType.DMA,
    ],
    compiler_params=pltpu.CompilerParams(
        needs_layout_passes=False,
    ),
)
def _gather_rows(x_hbm, y_hbm, out_hbm, rows_v, idx_v, out_v, x_sems, y_sem, out_sem):
    wid = lax.axis_index("s") * _NC + lax.axis_index("c")
    row0 = wid * ROWS_PER_W

    cy = pltpu.make_async_copy(y_hbm.at[pl.ds(row0, ROWS_PER_W)], idx_v, y_sem)
    cy.start()
    x_copies = []
    for r in range(ROWS_PER_W):
        cx = pltpu.make_async_copy(x_hbm.at[row0 + r], rows_v[r], x_sems[r])
        cx.start()
        x_copies.append(cx)
    cy.wait()

    for r in range(ROWS_PER_W):
        x_copies[r].wait()
        row_v = rows_v[r]

        @plsc.parallel_loop(0, B // L, unroll=8)
        def _(j):
            base = j * L
            idx = idx_v[r, pl.ds(base, L)]
            out_v[r, pl.ds(base, L)] = plsc.load_gather(row_v, [idx])

    co = pltpu.make_async_copy(out_v, out_hbm.at[pl.ds(row0, ROWS_PER_W)], out_sem)
    co.start()
    co.wait()


def kernel(x, y):
    return _gather_rows(x, y.astype(jnp.int32))
